# baseline (device time: 192748 ns/iter reference)
import jax
import jax.numpy as jnp
from jax import lax
from jax.experimental import pallas as pl
from jax.experimental.pallas import tpu as pltpu

S_HALF = 1024
K = 4096
N = 8192
NH = N // 2
TN = 512
T = NH // TN
NSLOT = 5
CH = 128

DOT_DIMS = (((1,), (0,)), ((), ()))


def kernel(O, Wo):
    my_y = lax.axis_index("y")
    bases = jnp.stack([my_y * T, (1 - my_y) * T]).astype(jnp.int32)

    def body(s_ref, o_ref, w_ref, out_ref,
             a_scr, stage, sendx, recvx, stash, sendy, recvy,
             stage_sems, sx_sems, rx_sems, sy_sems, ry_sems,
             creditx, credity):
        t = pl.program_id(0)
        p = pl.program_id(1)
        x = lax.axis_index("x")
        y = lax.axis_index("y")
        xnbr = (1 - x, y)
        ynbr = (x, 1 - y)

        def mk_x(j):
            return pltpu.make_async_remote_copy(
                src_ref=sendx.at[lax.rem(j, 2)],
                dst_ref=recvx.at[lax.rem(j, NSLOT)],
                send_sem=sx_sems.at[lax.rem(j, 2)],
                recv_sem=rx_sems.at[lax.rem(j, NSLOT)],
                device_id=xnbr,
                device_id_type=pl.DeviceIdType.MESH,
            )

        def mk_y(j):
            return pltpu.make_async_remote_copy(
                src_ref=sendy.at[lax.rem(j, 2)],
                dst_ref=recvy.at[lax.rem(j, NSLOT)],
                send_sem=sy_sems.at[lax.rem(j, 2)],
                recv_sem=ry_sems.at[lax.rem(j, NSLOT)],
                device_id=ynbr,
                device_id_type=pl.DeviceIdType.MESH,
            )

        r_nbr = (1 - x) * S_HALF
        r_own = x * S_HALF

        NHEADS = K // 128

        def prep_dma(h):
            return pltpu.make_async_copy(
                o_ref.at[0, :, h, :],
                stage.at[h % 2],
                stage_sems.at[h % 2],
            )

        @pl.when(jnp.logical_and(t == 0, p == 0))
        def _():
            bar = pltpu.get_barrier_semaphore()
            for nbr in (xnbr, ynbr):
                pl.semaphore_signal(
                    bar, inc=1, device_id=nbr,
                    device_id_type=pl.DeviceIdType.MESH,
                )
            pl.semaphore_wait(bar, 2)

            prep_dma(0).start()
            prep_dma(1).start()
            for h in range(NHEADS):
                prep_dma(h).wait()
                a_scr[:, h * 128:(h + 1) * 128] = (
                    stage[h % 2].astype(jnp.bfloat16)
                )
                if h + 2 < NHEADS:
                    prep_dma(h + 2).start()

        @pl.when(p == 0)
        def _():
            @pl.when(jnp.logical_and(t >= 2, t <= T + 1))
            def _():
                mk_x(t - 2).wait_send()

            @pl.when(t < T)
            def _():
                w_bf = w_ref[...].astype(jnp.bfloat16)
                e = lax.dot_general(a_scr[pl.ds(r_nbr, S_HALF), :], w_bf,
                                    DOT_DIMS,
                                    preferred_element_type=jnp.float32)
                sendx[lax.rem(t, 2)] = e.astype(jnp.bfloat16)

                @pl.when(t >= NSLOT)
                def _():
                    pl.semaphore_wait(creditx, 1)

                mk_x(t).start()
                stash[lax.rem(t, 2)] = lax.dot_general(
                    a_scr[pl.ds(r_own, S_HALF), :], w_bf, DOT_DIMS,
                    preferred_element_type=jnp.float32,
                ).astype(jnp.bfloat16)

            @pl.when(jnp.logical_and(t >= 3, t <= T + 1))
            def _():
                mk_y(t - 3).wait_send()

            @pl.when(t == T + 1)
            def _():
                mk_y(T - 1).wait_send()

            @pl.when(jnp.logical_and(t >= 1, t <= T))
            def _():
                j = t - 1
                mk_x(j).wait_recv()
                ssum = (stash[lax.rem(j, 2)].astype(jnp.float32)
                        + recvx[lax.rem(j, NSLOT)].astype(jnp.float32))
                out_ref[0] = ssum

                @pl.when(j < T - NSLOT)
                def _():
                    pl.semaphore_signal(
                        creditx, inc=1, device_id=xnbr,
                        device_id_type=pl.DeviceIdType.MESH,
                    )

                sendy[lax.rem(j, 2)] = ssum.astype(jnp.bfloat16)

                @pl.when(j >= NSLOT)
                def _():
                    pl.semaphore_wait(credity, 1)

                mk_y(j).start()

        @pl.when(jnp.logical_and(p == 1, t >= 2))
        def _():
            j = t - 2
            mk_y(j).wait_recv()
            out_ref[0] = recvy[lax.rem(j, NSLOT)].astype(jnp.float32)

            @pl.when(j < T - NSLOT)
            def _():
                pl.semaphore_signal(
                    credity, inc=1, device_id=ynbr,
                    device_id_type=pl.DeviceIdType.MESH,
                )

    def out_idx(t, p, s):
        in_x_phase = jnp.logical_and(p == 0, t <= T)
        idx = jnp.where(in_x_phase,
                        s[0] + jnp.clip(t - 1, 0, T - 1),
                        s[1] + jnp.clip(t - 2, 0, T - 1))
        return (0, 0, idx)

    out = pl.pallas_call(
        body,
        grid_spec=pltpu.PrefetchScalarGridSpec(
            num_scalar_prefetch=1,
            grid=(T + 2, 2),
            in_specs=[
                pl.BlockSpec(memory_space=pl.ANY),
                pl.BlockSpec((K, TN),
                             lambda t, p, s: (0, s[0] + jnp.clip(t, 0, T - 1))),
            ],
            out_specs=pl.BlockSpec((1, S_HALF, TN), out_idx),
            scratch_shapes=[
                pltpu.VMEM((2 * S_HALF, K), jnp.bfloat16),
                pltpu.VMEM((2, 2 * S_HALF, 128), jnp.float32),
                pltpu.VMEM((2, S_HALF, TN), jnp.bfloat16),
                pltpu.VMEM((NSLOT, S_HALF, TN), jnp.bfloat16),
                pltpu.VMEM((2, S_HALF, TN), jnp.bfloat16),
                pltpu.VMEM((2, S_HALF, TN), jnp.bfloat16),
                pltpu.VMEM((NSLOT, S_HALF, TN), jnp.bfloat16),
                pltpu.SemaphoreType.DMA((2,)),
                pltpu.SemaphoreType.DMA((2,)),
                pltpu.SemaphoreType.DMA((NSLOT,)),
                pltpu.SemaphoreType.DMA((2,)),
                pltpu.SemaphoreType.DMA((NSLOT,)),
                pltpu.SemaphoreType.REGULAR,
                pltpu.SemaphoreType.REGULAR,
            ],
        ),
        out_shape=jax.ShapeDtypeStruct((1, S_HALF, N), jnp.float32),
        compiler_params=pltpu.CompilerParams(
            collective_id=0,
            dimension_semantics=("arbitrary", "arbitrary"),
            vmem_limit_bytes=100 * 1024 * 1024,
        ),
    )(bases, O, Wo)
    return out


# device time: 184509 ns/iter; 1.0447x vs baseline; 1.0447x over previous
import jax
import jax.numpy as jnp
from jax import lax
from jax.experimental import pallas as pl
from jax.experimental.pallas import tpu as pltpu

S_HALF = 1024
K = 4096
N = 8192
NH = N // 2
TN = 512
T = NH // TN
NSLOT = 5
CH = 128

DOT_DIMS = (((1,), (0,)), ((), ()))


def kernel(O, Wo):
    my_y = lax.axis_index("y")
    bases = jnp.stack([my_y * T, (1 - my_y) * T]).astype(jnp.int32)

    def body(s_ref, o_ref, w_ref, out_ref,
             a_nbr, a_own, stage, sendx, recvx, stash, sendy, recvy,
             stage_sems, sx_sems, rx_sems, sy_sems, ry_sems,
             creditx, credity):
        t = pl.program_id(0)
        p = pl.program_id(1)
        x = lax.axis_index("x")
        y = lax.axis_index("y")
        xnbr = (1 - x, y)
        ynbr = (x, 1 - y)

        def mk_x(j):
            return pltpu.make_async_remote_copy(
                src_ref=sendx.at[lax.rem(j, 2)],
                dst_ref=recvx.at[lax.rem(j, NSLOT)],
                send_sem=sx_sems.at[lax.rem(j, 2)],
                recv_sem=rx_sems.at[lax.rem(j, NSLOT)],
                device_id=xnbr,
                device_id_type=pl.DeviceIdType.MESH,
            )

        def mk_y(j):
            return pltpu.make_async_remote_copy(
                src_ref=sendy.at[lax.rem(j, 2)],
                dst_ref=recvy.at[lax.rem(j, NSLOT)],
                send_sem=sy_sems.at[lax.rem(j, 2)],
                recv_sem=ry_sems.at[lax.rem(j, NSLOT)],
                device_id=ynbr,
                device_id_type=pl.DeviceIdType.MESH,
            )

        NPREP = S_HALF // CH
        r_nbr = (1 - x) * S_HALF
        r_own = x * S_HALF
        jobs = [(a_nbr, r_nbr, c) for c in range(NPREP)] + \
               [(a_own, r_own, c) for c in range(NPREP)]

        def prep_dma(i):
            dst, base, c = jobs[i]
            return pltpu.make_async_copy(
                o_ref.at[0, pl.ds(base + c * CH, CH), :, :],
                stage.at[i % 2],
                stage_sems.at[i % 2],
            )

        def prep_cast(lo, hi):
            for i in range(lo, hi):
                dst, base, c = jobs[i]
                prep_dma(i).wait()
                dst[pl.ds(c * CH, CH), :] = (
                    stage[i % 2].astype(jnp.bfloat16).reshape(CH, K)
                )
                if i + 2 < len(jobs):
                    prep_dma(i + 2).start()

        @pl.when(jnp.logical_and(t == 0, p == 0))
        def _():
            bar = pltpu.get_barrier_semaphore()
            for nbr in (xnbr, ynbr):
                pl.semaphore_signal(
                    bar, inc=1, device_id=nbr,
                    device_id_type=pl.DeviceIdType.MESH,
                )
            pl.semaphore_wait(bar, 2)

            prep_dma(0).start()
            prep_dma(1).start()
            prep_cast(0, NPREP)

        @pl.when(p == 0)
        def _():
            @pl.when(jnp.logical_and(t >= 2, t <= T + 1))
            def _():
                mk_x(t - 2).wait_send()

            @pl.when(t < T)
            def _():
                w_bf = w_ref[...].astype(jnp.bfloat16)
                e = lax.dot_general(a_nbr[...], w_bf, DOT_DIMS,
                                    preferred_element_type=jnp.float32)
                sendx[lax.rem(t, 2)] = e.astype(jnp.bfloat16)

                @pl.when(t >= NSLOT)
                def _():
                    pl.semaphore_wait(creditx, 1)

                mk_x(t).start()

                @pl.when(t == 0)
                def _():
                    prep_cast(NPREP, 2 * NPREP)

                stash[lax.rem(t, 2)] = lax.dot_general(
                    a_own[...], w_bf, DOT_DIMS,
                    preferred_element_type=jnp.float32,
                ).astype(jnp.bfloat16)

            @pl.when(jnp.logical_and(t >= 3, t <= T + 1))
            def _():
                mk_y(t - 3).wait_send()

            @pl.when(t == T + 1)
            def _():
                mk_y(T - 1).wait_send()

            @pl.when(jnp.logical_and(t >= 1, t <= T))
            def _():
                j = t - 1
                mk_x(j).wait_recv()
                ssum = (stash[lax.rem(j, 2)].astype(jnp.float32)
                        + recvx[lax.rem(j, NSLOT)].astype(jnp.float32))
                out_ref[0] = ssum

                @pl.when(j < T - NSLOT)
                def _():
                    pl.semaphore_signal(
                        creditx, inc=1, device_id=xnbr,
                        device_id_type=pl.DeviceIdType.MESH,
                    )

                sendy[lax.rem(j, 2)] = ssum.astype(jnp.bfloat16)

                @pl.when(j >= NSLOT)
                def _():
                    pl.semaphore_wait(credity, 1)

                mk_y(j).start()

        @pl.when(jnp.logical_and(p == 1, t >= 2))
        def _():
            j = t - 2
            mk_y(j).wait_recv()
            out_ref[0] = recvy[lax.rem(j, NSLOT)].astype(jnp.float32)

            @pl.when(j < T - NSLOT)
            def _():
                pl.semaphore_signal(
                    credity, inc=1, device_id=ynbr,
                    device_id_type=pl.DeviceIdType.MESH,
                )

    def out_idx(t, p, s):
        in_x_phase = jnp.logical_and(p == 0, t <= T)
        idx = jnp.where(in_x_phase,
                        s[0] + jnp.clip(t - 1, 0, T - 1),
                        s[1] + jnp.clip(t - 2, 0, T - 1))
        return (0, 0, idx)

    out = pl.pallas_call(
        body,
        grid_spec=pltpu.PrefetchScalarGridSpec(
            num_scalar_prefetch=1,
            grid=(T + 2, 2),
            in_specs=[
                pl.BlockSpec(memory_space=pl.ANY),
                pl.BlockSpec((K, TN),
                             lambda t, p, s: (0, s[0] + jnp.clip(t, 0, T - 1))),
            ],
            out_specs=pl.BlockSpec((1, S_HALF, TN), out_idx),
            scratch_shapes=[
                pltpu.VMEM((S_HALF, K), jnp.bfloat16),
                pltpu.VMEM((S_HALF, K), jnp.bfloat16),
                pltpu.VMEM((2, CH, K // 128, 128), jnp.float32),
                pltpu.VMEM((2, S_HALF, TN), jnp.bfloat16),
                pltpu.VMEM((NSLOT, S_HALF, TN), jnp.bfloat16),
                pltpu.VMEM((2, S_HALF, TN), jnp.bfloat16),
                pltpu.VMEM((2, S_HALF, TN), jnp.bfloat16),
                pltpu.VMEM((NSLOT, S_HALF, TN), jnp.bfloat16),
                pltpu.SemaphoreType.DMA((2,)),
                pltpu.SemaphoreType.DMA((2,)),
                pltpu.SemaphoreType.DMA((NSLOT,)),
                pltpu.SemaphoreType.DMA((2,)),
                pltpu.SemaphoreType.DMA((NSLOT,)),
                pltpu.SemaphoreType.REGULAR,
                pltpu.SemaphoreType.REGULAR,
            ],
        ),
        out_shape=jax.ShapeDtypeStruct((1, S_HALF, N), jnp.float32),
        compiler_params=pltpu.CompilerParams(
            collective_id=0,
            dimension_semantics=("arbitrary", "arbitrary"),
            vmem_limit_bytes=100 * 1024 * 1024,
        ),
    )(bases, O, Wo)
    return out
